# baseline (speedup 1.0000x reference)
import functools

import jax
import jax.numpy as jnp
from jax import lax
from jax.experimental import pallas as pl
from jax.experimental.pallas import tpu as pltpu
from jax.experimental.pallas import tpu_sc as plsc

_MASK_RATIO = 0.75

_IC = 64
_JC = 256


def _rank_body(noise_row_ref, noise_col_ref, rank_ref, mask_ref, *, L, len_keep):
    ic = pl.program_id(1)
    ki = noise_col_ref[0]
    diag_jc = (ic * _IC) // _JC

    def body(jc, acc):
        kj = noise_row_ref[0, :, pl.ds(jc * _JC, _JC)]

        def before(a):
            return a + jnp.where(ki < kj, 0.0, 1.0)

        def diag(a):
            jidx = jc * _JC + lax.broadcasted_iota(jnp.int32, (_IC, _JC), 1)
            iidx = ic * _IC + lax.broadcasted_iota(jnp.int32, (_IC, _JC), 0)
            cond = (kj < ki) | ((kj == ki) & (jidx < iidx))
            return a + jnp.where(cond, 1.0, 0.0)

        def after(a):
            return a + jnp.where(kj < ki, 1.0, 0.0)

        which = jnp.where(jc < diag_jc, 0, jnp.where(jc == diag_jc, 1, 2))
        return lax.switch(which, [before, diag, after], acc)

    acc = lax.fori_loop(0, L // _JC, body, jnp.zeros((_IC, _JC), jnp.float32))
    rank = jnp.sum(acc, axis=1, keepdims=True).astype(jnp.int32)
    rank_ref[0] = rank
    mask_ref[0] = (rank >= len_keep).astype(jnp.float32)


def _compute_rank(noise, interpret=False):
    N, L = noise.shape
    len_keep = int(L * (1 - _MASK_RATIO))
    body = functools.partial(_rank_body, L=L, len_keep=len_keep)
    rank3, mask3 = pl.pallas_call(
        body,
        grid=(N, L // _IC),
        in_specs=[
            pl.BlockSpec((1, 1, L), lambda r, ic: (r, 0, 0)),
            pl.BlockSpec((1, _IC, 1), lambda r, ic: (r, ic, 0)),
        ],
        out_specs=[
            pl.BlockSpec((1, _IC, 1), lambda r, ic: (r, ic, 0)),
            pl.BlockSpec((1, _IC, 1), lambda r, ic: (r, ic, 0)),
        ],
        out_shape=[
            jax.ShapeDtypeStruct((N, L, 1), jnp.int32),
            jax.ShapeDtypeStruct((N, L, 1), jnp.float32),
        ],
        interpret=interpret,
    )(noise.reshape(N, 1, L), noise.reshape(N, L, 1))
    return rank3.reshape(N, L), mask3.reshape(N, L)


_CH = 512


def _make_sc_gather(N, F, L, D, len_keep):
    n_rows_out = N * F * len_keep
    workers = 32
    rows_per_w = n_rows_out // workers
    f_per_w = rows_per_w // len_keep
    w_per_n = F // f_per_w
    chunks = rows_per_w // _CH
    mesh = plsc.VectorSubcoreMesh(core_axis_name="c", subcore_axis_name="s")

    @functools.partial(
        pl.kernel,
        mesh=mesh,
        out_type=jax.ShapeDtypeStruct((n_rows_out, D), jnp.float32),
        scratch_types=[
            pltpu.VMEM((L,), jnp.int32),
            pltpu.VMEM((len_keep,), jnp.int32),
            pltpu.VMEM((_CH,), jnp.int32),
            pltpu.VMEM((_CH, D), jnp.float32),
            pltpu.SemaphoreType.DMA,
        ],
        compiler_params=pltpu.CompilerParams(
            needs_layout_passes=False, use_tc_tiling_on_sc=False
        ),
    )
    def sc_gather(rank_hbm, x_hbm, y_hbm, rank_v, keep_v, idx_v, data_v, sem):
        c = lax.axis_index("c")
        s = lax.axis_index("s")
        w = s * 2 + c
        n = w // w_per_n
        f0 = (w % w_per_n) * f_per_w

        pltpu.sync_copy(rank_hbm.at[n], rank_v)

        def inv_body(i, carry):
            rk = rank_v[pl.ds(i * 16, 16)]
            vals = i * 16 + lax.iota(jnp.int32, 16)
            keepm = rk < len_keep
            idxc = jnp.where(keepm, rk, len_keep - 1)
            plsc.store_scatter(keep_v, [idxc], vals, mask=keepm)
            return carry

        lax.fori_loop(0, L // 16, inv_body, 0)

        def ch_body(t, carry):
            f = f0 + t // (len_keep // _CH)
            koff = (t % (len_keep // _CH)) * _CH
            base = (n * F + f) * L

            def idx_body(q, carry2):
                kp = keep_v[pl.ds(koff + q * 16, 16)]
                idx_v[pl.ds(q * 16, 16)] = kp + base
                return carry2

            lax.fori_loop(0, _CH // 16, idx_body, 0)
            pltpu.async_copy(x_hbm.at[idx_v], data_v, sem).wait()
            out_base = w * rows_per_w + t * _CH
            pltpu.sync_copy(data_v, y_hbm.at[pl.ds(out_base, _CH)])
            return carry

        lax.fori_loop(0, chunks, ch_body, 0)

    return sc_gather


def kernel(x):
    N, F, L, D = x.shape
    len_keep = int(L * (1 - _MASK_RATIO))
    noise = jax.random.uniform(jax.random.key(42), (N, L), dtype=x.dtype)
    rank, mask_row = _compute_rank(noise)
    sc_gather = _make_sc_gather(N, F, L, D, len_keep)
    y = sc_gather(rank, x.reshape(N * F * L, D))
    x_masked = y.reshape(N, F, len_keep, D)
    mask = jnp.broadcast_to(mask_row[:, None, :], (N, F, L))
    ids_restore = jnp.broadcast_to(rank[:, None, :], (N, F, L))
    return (x_masked, mask, ids_restore)


# speedup vs baseline: 2.2850x; 2.2850x over previous
import functools

import jax
import jax.numpy as jnp
from jax import lax
from jax.experimental import pallas as pl
from jax.experimental.pallas import tpu as pltpu
from jax.experimental.pallas import tpu_sc as plsc

_MASK_RATIO = 0.75

_IC = 128
_JC = 128


def _rank_body(noise_row_ref, noise_col_ref, rank_ref, mask_ref, *, L, len_keep):
    ic = pl.program_id(1)
    nj = L // _JC
    ki = noise_col_ref[0]
    ki_b = jnp.broadcast_to(ki, (_IC, _JC))

    def kj_at(jc):
        return noise_row_ref[0, :, pl.ds(jc * _JC, _JC)]

    def before(jc, acc):
        return acc + jnp.where(ki_b < kj_at(jc), -1.0, 0.0)

    def after(jc, acc):
        return acc + jnp.where(kj_at(jc) < ki_b, 1.0, 0.0)

    acc = lax.fori_loop(0, ic, before, jnp.zeros((_IC, _JC), jnp.float32))
    acc = lax.fori_loop(ic + 1, nj, after, acc)
    kj = kj_at(ic)
    jlt = (lax.broadcasted_iota(jnp.int32, (_IC, _JC), 1)
           < lax.broadcasted_iota(jnp.int32, (_IC, _JC), 0))
    cond = (kj < ki_b) | ((kj == ki_b) & jlt)
    acc = acc + jnp.where(cond, 1.0, 0.0)

    rank = (ic * _JC
            + jnp.sum(acc, axis=1, keepdims=True).astype(jnp.int32))
    rank_ref[0] = rank
    mask_ref[0] = (rank >= len_keep).astype(jnp.float32)


def _compute_rank(noise, interpret=False):
    N, L = noise.shape
    len_keep = int(L * (1 - _MASK_RATIO))
    body = functools.partial(_rank_body, L=L, len_keep=len_keep)
    rank3, mask3 = pl.pallas_call(
        body,
        grid=(N, L // _IC),
        in_specs=[
            pl.BlockSpec((1, 1, L), lambda r, ic: (r, 0, 0)),
            pl.BlockSpec((1, _IC, 1), lambda r, ic: (r, ic, 0)),
        ],
        out_specs=[
            pl.BlockSpec((1, _IC, 1), lambda r, ic: (r, ic, 0)),
            pl.BlockSpec((1, _IC, 1), lambda r, ic: (r, ic, 0)),
        ],
        out_shape=[
            jax.ShapeDtypeStruct((N, L, 1), jnp.int32),
            jax.ShapeDtypeStruct((N, L, 1), jnp.float32),
        ],
        interpret=interpret,
    )(noise.reshape(N, 1, L), noise.reshape(N, L, 1))
    return rank3.reshape(N, L), mask3.reshape(N, L)


_CH = 512


def _make_sc_gather(N, F, L, D, len_keep):
    n_rows_out = N * F * len_keep
    workers = 32
    rows_per_w = n_rows_out // workers
    f_per_w = rows_per_w // len_keep
    w_per_n = F // f_per_w
    chunks = rows_per_w // _CH
    mesh = plsc.VectorSubcoreMesh(core_axis_name="c", subcore_axis_name="s")

    @functools.partial(
        pl.kernel,
        mesh=mesh,
        out_type=jax.ShapeDtypeStruct((n_rows_out, D), jnp.float32),
        scratch_types=[
            pltpu.VMEM((L,), jnp.int32),
            pltpu.VMEM((len_keep,), jnp.int32),
            pltpu.VMEM((_CH,), jnp.int32),
            pltpu.VMEM((_CH, D), jnp.float32),
            pltpu.SemaphoreType.DMA,
        ],
        compiler_params=pltpu.CompilerParams(
            needs_layout_passes=False, use_tc_tiling_on_sc=False
        ),
    )
    def sc_gather(rank_hbm, x_hbm, y_hbm, rank_v, keep_v, idx_v, data_v, sem):
        c = lax.axis_index("c")
        s = lax.axis_index("s")
        w = s * 2 + c
        n = w // w_per_n
        f0 = (w % w_per_n) * f_per_w

        pltpu.sync_copy(rank_hbm.at[n], rank_v)

        def inv_body(i, carry):
            rk = rank_v[pl.ds(i * 16, 16)]
            vals = i * 16 + lax.iota(jnp.int32, 16)
            keepm = rk < len_keep
            idxc = jnp.where(keepm, rk, len_keep - 1)
            plsc.store_scatter(keep_v, [idxc], vals, mask=keepm)
            return carry

        lax.fori_loop(0, L // 16, inv_body, 0)

        def ch_body(t, carry):
            f = f0 + t // (len_keep // _CH)
            koff = (t % (len_keep // _CH)) * _CH
            base = (n * F + f) * L

            def idx_body(q, carry2):
                kp = keep_v[pl.ds(koff + q * 16, 16)]
                idx_v[pl.ds(q * 16, 16)] = kp + base
                return carry2

            lax.fori_loop(0, _CH // 16, idx_body, 0)
            pltpu.async_copy(x_hbm.at[idx_v], data_v, sem).wait()
            out_base = w * rows_per_w + t * _CH
            pltpu.sync_copy(data_v, y_hbm.at[pl.ds(out_base, _CH)])
            return carry

        lax.fori_loop(0, chunks, ch_body, 0)

    return sc_gather


def kernel(x):
    N, F, L, D = x.shape
    len_keep = int(L * (1 - _MASK_RATIO))
    noise = jax.random.uniform(jax.random.key(42), (N, L), dtype=x.dtype)
    rank, mask_row = _compute_rank(noise)
    sc_gather = _make_sc_gather(N, F, L, D, len_keep)
    y = sc_gather(rank, x.reshape(N * F * L, D))
    x_masked = y.reshape(N, F, len_keep, D)
    mask = jnp.broadcast_to(mask_row[:, None, :], (N, F, L))
    ids_restore = jnp.broadcast_to(rank[:, None, :], (N, F, L))
    return (x_masked, mask, ids_restore)


# speedup vs baseline: 2.3319x; 1.0206x over previous
import functools

import jax
import jax.numpy as jnp
from jax import lax
from jax.experimental import pallas as pl
from jax.experimental.pallas import tpu as pltpu
from jax.experimental.pallas import tpu_sc as plsc

_MASK_RATIO = 0.75

_IC = 128
_JC = 128


def _rank_body(noise_row_ref, noise_col_ref, rank_ref, mask_ref, *, L, len_keep):
    ic = pl.program_id(1)
    nj = L // _JC
    ki = noise_col_ref[0]
    ki_b = jnp.broadcast_to(ki, (_IC, _JC))

    def kj_at(jc):
        return noise_row_ref[0, :, pl.ds(jc * _JC, _JC)]

    def before(jc, acc):
        return acc + jnp.where(ki_b < kj_at(jc), -1.0, 0.0)

    def after(jc, acc):
        return acc + jnp.where(kj_at(jc) < ki_b, 1.0, 0.0)

    U = 4

    def before_u(t, acc):
        for u in range(U):
            acc = before(t * U + u, acc)
        return acc

    def after_u(t, acc):
        for u in range(U):
            acc = after(ic + 1 + t * U + u, acc)
        return acc

    acc = jnp.zeros((_IC, _JC), jnp.float32)
    acc = lax.fori_loop(0, ic // U, before_u, acc)
    acc = lax.fori_loop((ic // U) * U, ic, before, acc)
    n_after = nj - ic - 1
    acc = lax.fori_loop(0, n_after // U, after_u, acc)
    acc = lax.fori_loop(ic + 1 + (n_after // U) * U, nj, after, acc)
    kj = kj_at(ic)
    jlt = (lax.broadcasted_iota(jnp.int32, (_IC, _JC), 1)
           < lax.broadcasted_iota(jnp.int32, (_IC, _JC), 0))
    cond = (kj < ki_b) | ((kj == ki_b) & jlt)
    acc = acc + jnp.where(cond, 1.0, 0.0)

    rank = (ic * _JC
            + jnp.sum(acc, axis=1, keepdims=True).astype(jnp.int32))
    rank_ref[0] = rank
    mask_ref[0] = (rank >= len_keep).astype(jnp.float32)


def _compute_rank(noise, interpret=False):
    N, L = noise.shape
    len_keep = int(L * (1 - _MASK_RATIO))
    body = functools.partial(_rank_body, L=L, len_keep=len_keep)
    rank3, mask3 = pl.pallas_call(
        body,
        grid=(N, L // _IC),
        in_specs=[
            pl.BlockSpec((1, 1, L), lambda r, ic: (r, 0, 0)),
            pl.BlockSpec((1, _IC, 1), lambda r, ic: (r, ic, 0)),
        ],
        out_specs=[
            pl.BlockSpec((1, _IC, 1), lambda r, ic: (r, ic, 0)),
            pl.BlockSpec((1, _IC, 1), lambda r, ic: (r, ic, 0)),
        ],
        out_shape=[
            jax.ShapeDtypeStruct((N, L, 1), jnp.int32),
            jax.ShapeDtypeStruct((N, L, 1), jnp.float32),
        ],
        interpret=interpret,
    )(noise.reshape(N, 1, L), noise.reshape(N, L, 1))
    return rank3.reshape(N, L), mask3.reshape(N, L)


_CH = 512


def _make_sc_gather(N, F, L, D, len_keep):
    n_rows_out = N * F * len_keep
    workers = 32
    rows_per_w = n_rows_out // workers
    f_per_w = rows_per_w // len_keep
    w_per_n = F // f_per_w
    chunks = rows_per_w // _CH
    mesh = plsc.VectorSubcoreMesh(core_axis_name="c", subcore_axis_name="s")

    @functools.partial(
        pl.kernel,
        mesh=mesh,
        out_type=jax.ShapeDtypeStruct((n_rows_out, D), jnp.float32),
        scratch_types=[
            pltpu.VMEM((L,), jnp.int32),
            pltpu.VMEM((len_keep,), jnp.int32),
            pltpu.VMEM((_CH,), jnp.int32),
            pltpu.VMEM((_CH, D), jnp.float32),
            pltpu.SemaphoreType.DMA,
        ],
        compiler_params=pltpu.CompilerParams(
            needs_layout_passes=False, use_tc_tiling_on_sc=False
        ),
    )
    def sc_gather(rank_hbm, x_hbm, y_hbm, rank_v, keep_v, idx_v, data_v, sem):
        c = lax.axis_index("c")
        s = lax.axis_index("s")
        w = s * 2 + c
        n = w // w_per_n
        f0 = (w % w_per_n) * f_per_w

        pltpu.sync_copy(rank_hbm.at[n], rank_v)

        def inv_body(i, carry):
            rk = rank_v[pl.ds(i * 16, 16)]
            vals = i * 16 + lax.iota(jnp.int32, 16)
            keepm = rk < len_keep
            idxc = jnp.where(keepm, rk, len_keep - 1)
            plsc.store_scatter(keep_v, [idxc], vals, mask=keepm)
            return carry

        lax.fori_loop(0, L // 16, inv_body, 0)

        def ch_body(t, carry):
            f = f0 + t // (len_keep // _CH)
            koff = (t % (len_keep // _CH)) * _CH
            base = (n * F + f) * L

            def idx_body(q, carry2):
                kp = keep_v[pl.ds(koff + q * 16, 16)]
                idx_v[pl.ds(q * 16, 16)] = kp + base
                return carry2

            lax.fori_loop(0, _CH // 16, idx_body, 0)
            pltpu.async_copy(x_hbm.at[idx_v], data_v, sem).wait()
            out_base = w * rows_per_w + t * _CH
            pltpu.sync_copy(data_v, y_hbm.at[pl.ds(out_base, _CH)])
            return carry

        lax.fori_loop(0, chunks, ch_body, 0)

    return sc_gather


def kernel(x):
    N, F, L, D = x.shape
    len_keep = int(L * (1 - _MASK_RATIO))
    noise = jax.random.uniform(jax.random.key(42), (N, L), dtype=x.dtype)
    rank, mask_row = _compute_rank(noise)
    sc_gather = _make_sc_gather(N, F, L, D, len_keep)
    y = sc_gather(rank, x.reshape(N * F * L, D))
    x_masked = y.reshape(N, F, len_keep, D)
    mask = jnp.broadcast_to(mask_row[:, None, :], (N, F, L))
    ids_restore = jnp.broadcast_to(rank[:, None, :], (N, F, L))
    return (x_masked, mask, ids_restore)
